# baseline (speedup 1.0000x reference)
import functools

import jax
import jax.numpy as jnp
from jax import lax
from jax.experimental import pallas as pl
from jax.experimental.pallas import tpu as pltpu
from jax.experimental.pallas import tpu_sc as plsc

N = 10000
E = 320000
D = 128
NC, NS = 2, 16
NW = NC * NS
CHUNK = 32
K0, K1 = 460, 180
E_PAD = NS * (K0 + K1) * CHUNK
TOT_ROWS = NS * (K0 + K1) + K0 + 2
N_PAD = 10016
W1 = D + 16
ROWS_PER_TILE = N_PAD // NS


@functools.lru_cache(maxsize=None)
def _sc_segment_sum(w):
  mesh = plsc.VectorSubcoreMesh(
      core_axis_name="c", subcore_axis_name="s", num_cores=NC,
      num_subcores=NS)

  @functools.partial(
      pl.kernel,
      out_type=jax.ShapeDtypeStruct((NC, N_PAD, w), jnp.float32),
      mesh=mesh,
      compiler_params=pltpu.CompilerParams(use_tc_tiling_on_sc=False),
      scratch_types=[
          pltpu.VMEM((K0 + 2, CHUNK), jnp.int32),
          pltpu.VMEM((K0, CHUNK), jnp.int32),
          pltpu.VMEM((CHUNK, w), jnp.float32),
          pltpu.VMEM((CHUNK, w), jnp.float32),
          pltpu.VMEM_SHARED((N_PAD, w), jnp.float32),
          pltpu.SemaphoreType.DMA,
          pltpu.SemaphoreType.DMA,
      ],
  )
  def k(tab_hbm, src_hbm, dst_hbm, zeros_hbm, out_hbm, src_v, dst_v,
        rows0, rows1, agg_sh, gsem0, gsem1):
    c = lax.axis_index("c")
    s = lax.axis_index("s")
    row0 = s * ROWS_PER_TILE
    base = jnp.where(c == 0, s * K0, NS * K0 + s * K1)
    nch = jnp.where(c == 0, K0, K1)

    pltpu.sync_copy(zeros_hbm, agg_sh.at[pl.ds(row0, ROWS_PER_TILE)])
    pltpu.sync_copy(src_hbm.at[pl.ds(base, K0 + 2)], src_v)
    pltpu.sync_copy(dst_hbm.at[pl.ds(base, K0)], dst_v)
    plsc.subcore_barrier()

    pltpu.async_copy(tab_hbm.at[src_v.at[0]], rows0, gsem0)
    pltpu.async_copy(tab_hbm.at[src_v.at[1]], rows1, gsem1)

    def body(jj, carry):
      j = jj * 2
      pltpu.make_async_copy(tab_hbm.at[src_v.at[j]], rows0, gsem0).wait()
      pltpu.sync_copy(rows0, agg_sh.at[dst_v.at[j]], add=True)
      pltpu.async_copy(tab_hbm.at[src_v.at[j + 2]], rows0, gsem0)
      pltpu.make_async_copy(tab_hbm.at[src_v.at[j + 1]], rows1, gsem1).wait()
      pltpu.sync_copy(rows1, agg_sh.at[dst_v.at[j + 1]], add=True)
      pltpu.async_copy(tab_hbm.at[src_v.at[j + 3]], rows1, gsem1)
      return carry
    lax.fori_loop(0, nch // 2, body, 0)
    pltpu.make_async_copy(tab_hbm.at[src_v.at[nch]], rows0, gsem0).wait()
    pltpu.make_async_copy(tab_hbm.at[src_v.at[nch + 1]], rows1, gsem1).wait()
    plsc.subcore_barrier()

    pltpu.sync_copy(agg_sh.at[pl.ds(row0, ROWS_PER_TILE)],
                    out_hbm.at[c].at[pl.ds(row0, ROWS_PER_TILE)])

  return k


def _kand(u, wcat):
  z0 = jnp.exp(-jnp.square((u + 1.0) * 0.5))
  z1 = jnp.exp(-jnp.square((u - 1.0) * 0.5))
  sil = u * jax.nn.sigmoid(u)
  feats = jnp.concatenate([z0, z1, sil], axis=1)
  return lax.dot_general(feats, wcat, (((1,), (0,)), ((), ())),
                         preferred_element_type=jnp.float32)


def _layernorm(t):
  mu = jnp.mean(t, axis=1, keepdims=True)
  xc = t - mu
  var = jnp.mean(xc * xc, axis=1, keepdims=True)
  return xc * lax.rsqrt(var + 1e-5)


R = 1000
TC_GRID = N // R


def _tc1_body(part_ref, x_ref, wcat_ref, out_ref):
  p = part_ref[0] + part_ref[1]
  deg = jnp.sum(p[:, D:W1], axis=1, keepdims=True) + 1.0
  u = (p[:, :D] + x_ref[...]) / deg
  out_ref[...] = _layernorm(_kand(u, wcat_ref[...]))


def _tc2_body(part1_ref, part2_ref, h1_ref, wcat1_ref, wcato_ref, out_ref):
  p1 = part1_ref[0] + part1_ref[1]
  deg = jnp.sum(p1[:, D:W1], axis=1, keepdims=True) + 1.0
  q = part2_ref[0] + part2_ref[1]
  u = (q + h1_ref[...]) / deg
  h2 = _layernorm(_kand(u, wcat1_ref[...]))
  out_ref[...] = _kand(h2, wcato_ref[...])


_tc1 = pl.pallas_call(
    _tc1_body,
    grid=(TC_GRID,),
    in_specs=[
        pl.BlockSpec((NC, R, W1), lambda i: (0, i, 0)),
        pl.BlockSpec((R, D), lambda i: (i, 0)),
        pl.BlockSpec((3 * D, D), lambda i: (0, 0)),
    ],
    out_specs=pl.BlockSpec((R, D), lambda i: (i, 0)),
    out_shape=jax.ShapeDtypeStruct((N, D), jnp.float32),
)

_tc2 = pl.pallas_call(
    _tc2_body,
    grid=(TC_GRID,),
    in_specs=[
        pl.BlockSpec((NC, R, W1), lambda i: (0, i, 0)),
        pl.BlockSpec((NC, R, D), lambda i: (0, i, 0)),
        pl.BlockSpec((R, D), lambda i: (i, 0)),
        pl.BlockSpec((3 * D, D), lambda i: (0, 0)),
        pl.BlockSpec((3 * D, D), lambda i: (0, 0)),
    ],
    out_specs=pl.BlockSpec((R, D), lambda i: (i, 0)),
    out_shape=jax.ShapeDtypeStruct((N, D), jnp.float32),
)


def _wcat(spline_w, base_w):
  return jnp.concatenate(
      [spline_w[:, 0::2].T, spline_w[:, 1::2].T, base_w.T], axis=0)


def kernel(x, edge_index, grid0, spline0, base0, grid1, spline1, base1,
           grid_out, spline_out, base_out):
  src = edge_index[0].astype(jnp.int32)
  dst = edge_index[1].astype(jnp.int32)
  fill = jnp.full((TOT_ROWS * CHUNK - E,), N, jnp.int32)
  src2d = jnp.concatenate([src, fill]).reshape(TOT_ROWS, CHUNK)
  dst2d = jnp.concatenate([dst, fill]).reshape(TOT_ROWS, CHUNK)

  wcat0 = _wcat(spline0, base0)
  wcat1 = _wcat(spline1, base1)
  wcato = _wcat(spline_out, base_out)

  tab1 = jnp.zeros((N_PAD, W1), jnp.float32)
  tab1 = tab1.at[:N, :D].set(x).at[:N, D].set(1.0)
  part1 = _sc_segment_sum(W1)(tab1, src2d, dst2d,
                              jnp.zeros((ROWS_PER_TILE, W1), jnp.float32))
  h1 = _tc1(part1, x, wcat0)

  tab2 = jnp.zeros((N_PAD, D), jnp.float32).at[:N].set(h1)
  part2 = _sc_segment_sum(D)(tab2, src2d, dst2d,
                             jnp.zeros((ROWS_PER_TILE, D), jnp.float32))
  return _tc2(part1, part2, h1, wcat1, wcato)


# speedup vs baseline: 1.0027x; 1.0027x over previous
import functools

import jax
import jax.numpy as jnp
from jax import lax
from jax.experimental import pallas as pl
from jax.experimental.pallas import tpu as pltpu
from jax.experimental.pallas import tpu_sc as plsc

N = 10000
E = 320000
D = 128
NC, NS = 2, 16
NW = NC * NS
CHUNK = 32
K0, K1 = 492, 148
E_PAD = NS * (K0 + K1) * CHUNK
TOT_ROWS = NS * (K0 + K1) + K0 + 2
N_PAD = 10016
W1 = D + 16
ROWS_PER_TILE = N_PAD // NS


@functools.lru_cache(maxsize=None)
def _sc_segment_sum(w):
  mesh = plsc.VectorSubcoreMesh(
      core_axis_name="c", subcore_axis_name="s", num_cores=NC,
      num_subcores=NS)

  @functools.partial(
      pl.kernel,
      out_type=jax.ShapeDtypeStruct((NC, N_PAD, w), jnp.float32),
      mesh=mesh,
      compiler_params=pltpu.CompilerParams(use_tc_tiling_on_sc=False),
      scratch_types=[
          pltpu.VMEM((K0 + 2, CHUNK), jnp.int32),
          pltpu.VMEM((K0, CHUNK), jnp.int32),
          pltpu.VMEM((CHUNK, w), jnp.float32),
          pltpu.VMEM((CHUNK, w), jnp.float32),
          pltpu.VMEM_SHARED((N_PAD, w), jnp.float32),
          pltpu.SemaphoreType.DMA,
          pltpu.SemaphoreType.DMA,
      ],
  )
  def k(tab_hbm, src_hbm, dst_hbm, zeros_hbm, out_hbm, src_v, dst_v,
        rows0, rows1, agg_sh, gsem0, gsem1):
    c = lax.axis_index("c")
    s = lax.axis_index("s")
    row0 = s * ROWS_PER_TILE
    base = jnp.where(c == 0, s * K0, NS * K0 + s * K1)
    nch = jnp.where(c == 0, K0, K1)

    pltpu.sync_copy(zeros_hbm, agg_sh.at[pl.ds(row0, ROWS_PER_TILE)])
    pltpu.sync_copy(src_hbm.at[pl.ds(base, K0 + 2)], src_v)
    pltpu.sync_copy(dst_hbm.at[pl.ds(base, K0)], dst_v)
    plsc.subcore_barrier()

    pltpu.async_copy(tab_hbm.at[src_v.at[0]], rows0, gsem0)
    pltpu.async_copy(tab_hbm.at[src_v.at[1]], rows1, gsem1)

    def body(jj, carry):
      j = jj * 2
      pltpu.make_async_copy(tab_hbm.at[src_v.at[j]], rows0, gsem0).wait()
      pltpu.sync_copy(rows0, agg_sh.at[dst_v.at[j]], add=True)
      pltpu.async_copy(tab_hbm.at[src_v.at[j + 2]], rows0, gsem0)
      pltpu.make_async_copy(tab_hbm.at[src_v.at[j + 1]], rows1, gsem1).wait()
      pltpu.sync_copy(rows1, agg_sh.at[dst_v.at[j + 1]], add=True)
      pltpu.async_copy(tab_hbm.at[src_v.at[j + 3]], rows1, gsem1)
      return carry
    lax.fori_loop(0, nch // 2, body, 0)
    pltpu.make_async_copy(tab_hbm.at[src_v.at[nch]], rows0, gsem0).wait()
    pltpu.make_async_copy(tab_hbm.at[src_v.at[nch + 1]], rows1, gsem1).wait()
    plsc.subcore_barrier()

    pltpu.sync_copy(agg_sh.at[pl.ds(row0, ROWS_PER_TILE)],
                    out_hbm.at[c].at[pl.ds(row0, ROWS_PER_TILE)])

  return k


def _kand(u, wcat):
  z0 = jnp.exp(-jnp.square((u + 1.0) * 0.5))
  z1 = jnp.exp(-jnp.square((u - 1.0) * 0.5))
  sil = u * jax.nn.sigmoid(u)
  feats = jnp.concatenate([z0, z1, sil], axis=1)
  return lax.dot_general(feats, wcat, (((1,), (0,)), ((), ())),
                         preferred_element_type=jnp.float32)


def _layernorm(t):
  mu = jnp.mean(t, axis=1, keepdims=True)
  xc = t - mu
  var = jnp.mean(xc * xc, axis=1, keepdims=True)
  return xc * lax.rsqrt(var + 1e-5)


R = 1000
TC_GRID = N // R


def _tc1_body(part_ref, x_ref, wcat_ref, out_ref):
  p = part_ref[0] + part_ref[1]
  deg = jnp.sum(p[:, D:W1], axis=1, keepdims=True) + 1.0
  u = (p[:, :D] + x_ref[...]) / deg
  out_ref[...] = _layernorm(_kand(u, wcat_ref[...]))


def _tc2_body(part1_ref, part2_ref, h1_ref, wcat1_ref, wcato_ref, out_ref):
  p1 = part1_ref[0] + part1_ref[1]
  deg = jnp.sum(p1[:, D:W1], axis=1, keepdims=True) + 1.0
  q = part2_ref[0] + part2_ref[1]
  u = (q + h1_ref[...]) / deg
  h2 = _layernorm(_kand(u, wcat1_ref[...]))
  out_ref[...] = _kand(h2, wcato_ref[...])


_tc1 = pl.pallas_call(
    _tc1_body,
    grid=(TC_GRID,),
    in_specs=[
        pl.BlockSpec((NC, R, W1), lambda i: (0, i, 0)),
        pl.BlockSpec((R, D), lambda i: (i, 0)),
        pl.BlockSpec((3 * D, D), lambda i: (0, 0)),
    ],
    out_specs=pl.BlockSpec((R, D), lambda i: (i, 0)),
    out_shape=jax.ShapeDtypeStruct((N, D), jnp.float32),
)

_tc2 = pl.pallas_call(
    _tc2_body,
    grid=(TC_GRID,),
    in_specs=[
        pl.BlockSpec((NC, R, W1), lambda i: (0, i, 0)),
        pl.BlockSpec((NC, R, D), lambda i: (0, i, 0)),
        pl.BlockSpec((R, D), lambda i: (i, 0)),
        pl.BlockSpec((3 * D, D), lambda i: (0, 0)),
        pl.BlockSpec((3 * D, D), lambda i: (0, 0)),
    ],
    out_specs=pl.BlockSpec((R, D), lambda i: (i, 0)),
    out_shape=jax.ShapeDtypeStruct((N, D), jnp.float32),
)


def _wcat(spline_w, base_w):
  return jnp.concatenate(
      [spline_w[:, 0::2].T, spline_w[:, 1::2].T, base_w.T], axis=0)


def kernel(x, edge_index, grid0, spline0, base0, grid1, spline1, base1,
           grid_out, spline_out, base_out):
  src = edge_index[0].astype(jnp.int32)
  dst = edge_index[1].astype(jnp.int32)
  fill = jnp.full((TOT_ROWS * CHUNK - E,), N, jnp.int32)
  src2d = jnp.concatenate([src, fill]).reshape(TOT_ROWS, CHUNK)
  dst2d = jnp.concatenate([dst, fill]).reshape(TOT_ROWS, CHUNK)

  wcat0 = _wcat(spline0, base0)
  wcat1 = _wcat(spline1, base1)
  wcato = _wcat(spline_out, base_out)

  tab1 = jnp.zeros((N_PAD, W1), jnp.float32)
  tab1 = tab1.at[:N, :D].set(x).at[:N, D].set(1.0)
  part1 = _sc_segment_sum(W1)(tab1, src2d, dst2d,
                              jnp.zeros((ROWS_PER_TILE, W1), jnp.float32))
  h1 = _tc1(part1, x, wcat0)

  tab2 = jnp.zeros((N_PAD, D), jnp.float32).at[:N].set(h1)
  part2 = _sc_segment_sum(D)(tab2, src2d, dst2d,
                             jnp.zeros((ROWS_PER_TILE, D), jnp.float32))
  return _tc2(part1, part2, h1, wcat1, wcato)
